# baseline (speedup 1.0000x reference)
import functools

import jax
import jax.numpy as jnp
from jax import lax
from jax.experimental import pallas as pl
from jax.experimental.pallas import tpu as pltpu
from jax.experimental.pallas import tpu_sc as plsc

L = 16
NC = 2
NS = 16
NW = NC * NS
CH = 16
NBI = 4
NBO = 2


def _allsum_vec(v):
    idx = lax.iota(jnp.int32, L)
    dnums = lax.GatherDimensionNumbers(
        offset_dims=(), collapsed_slice_dims=(0,), start_index_map=(0,))
    for d in (1, 2, 4, 8):
        v = v + lax.gather(
            v, (idx ^ d)[:, None], dimension_numbers=dnums, slice_sizes=(1,),
            mode=lax.GatherScatterMode.PROMISE_IN_BOUNDS)
    return v


def _rsqrt_vec(v):
    i = lax.bitcast_convert_type(v, jnp.int32)
    magic = jnp.full((L,), 0x5F3759DF, jnp.int32)
    y = lax.bitcast_convert_type(magic - (i >> 1), jnp.float32)
    for _ in range(3):
        y = y * (1.5 - 0.5 * v * y * y)
    return y


def _make_sc_kernel(B, S, V, D):
    T = B * S
    tpw = T // NW
    nch = tpw // CH
    nvec = D // L
    mesh = plsc.VectorSubcoreMesh(
        core_axis_name="c", subcore_axis_name="s", num_cores=NC, num_subcores=NS
    )

    @functools.partial(
        pl.kernel,
        out_type=jax.ShapeDtypeStruct((T, D), jnp.float32),
        mesh=mesh,
        scratch_types=[
            pltpu.VMEM((tpw,), jnp.int32),
            pltpu.VMEM((S + (-S) % 8, D), jnp.float32),
            pltpu.VMEM((D,), jnp.float32),
            pltpu.VMEM((D,), jnp.float32),
            [pltpu.VMEM((CH, D), jnp.float32)] * NBI,
            [pltpu.VMEM((CH, D), jnp.float32)] * NBO,
            [pltpu.SemaphoreType.DMA] * NBI,
            [pltpu.SemaphoreType.DMA] * NBO,
        ],
    )
    def k(x_ref, tok_ref, pos_ref, gamma_ref, beta_ref, out_ref,
          idx_v, pos_v, gamma_v, beta_v, in_bufs, out_bufs, gsems, osems):
        wid = lax.axis_index("s") * NC + lax.axis_index("c")
        base = wid * tpw

        pltpu.sync_copy(x_ref.at[pl.ds(base, tpw)], idx_v)
        pltpu.sync_copy(pos_ref.at[pl.ds(0, S + (-S) % 8)], pos_v)
        pltpu.sync_copy(gamma_ref, gamma_v)
        pltpu.sync_copy(beta_ref, beta_v)

        def issue_gather(c, b):
            pltpu.async_copy(
                tok_ref.at[idx_v.at[pl.ds(c * CH, CH)]], in_bufs[b], gsems[b]
            )

        for b in range(NBI):
            issue_gather(b, b)

        def compute_chunk(c, b_in, b_out):
            phase = lax.rem(base + c * CH, S)

            def row_body(r, _):
                p = phase + r
                p = jnp.where(p >= S, p - S, p)
                sacc = [jnp.zeros((L,), jnp.float32) for _ in range(6)]
                qacc = [jnp.zeros((L,), jnp.float32) for _ in range(6)]
                for j in range(nvec):
                    v = (in_bufs[b_in][r, pl.ds(j * L, L)]
                         + pos_v[p, pl.ds(j * L, L)])
                    in_bufs[b_in][r, pl.ds(j * L, L)] = v
                    sacc[j % 6] = sacc[j % 6] + v
                    qacc[j % 6] = qacc[j % 6] + v * v
                s = ((sacc[0] + sacc[1]) + (sacc[2] + sacc[3])) + (sacc[4] + sacc[5])
                q = ((qacc[0] + qacc[1]) + (qacc[2] + qacc[3])) + (qacc[4] + qacc[5])
                mean_v = _allsum_vec(s) * (1.0 / D)
                msq_v = _allsum_vec(q) * (1.0 / D)
                var_v = msq_v - mean_v * mean_v
                rstd = _rsqrt_vec(var_v + 1e-5)
                for j in range(nvec):
                    g = gamma_v[pl.ds(j * L, L)]
                    bt = beta_v[pl.ds(j * L, L)]
                    v = in_bufs[b_in][r, pl.ds(j * L, L)]
                    out_bufs[b_out][r, pl.ds(j * L, L)] = (
                        (v - mean_v) * (rstd * g) + bt
                    )
                return 0

            lax.fori_loop(0, CH, row_body, 0)

        def outer(o, _):
            for q in range(NBI):
                c = o * NBI + q
                b_in = q
                b_out = q % NBO
                @pl.when(c >= NBO)
                def _():
                    pltpu.make_async_copy(
                        out_bufs[b_out], out_ref.at[pl.ds(0, CH)], osems[b_out]
                    ).wait()
                pltpu.make_async_copy(
                    tok_ref.at[idx_v.at[pl.ds(c * CH, CH)]],
                    in_bufs[b_in], gsems[b_in],
                ).wait()
                compute_chunk(c, b_in, b_out)
                pltpu.async_copy(
                    out_bufs[b_out], out_ref.at[pl.ds(base + c * CH, CH)],
                    osems[b_out],
                )
                @pl.when(c + NBI < nch)
                def _():
                    issue_gather(c + NBI, b_in)
            return 0

        lax.fori_loop(0, nch // NBI, outer, 0)
        for b in range(NBO):
            pltpu.make_async_copy(
                out_bufs[b], out_ref.at[pl.ds(0, CH)], osems[b]
            ).wait()

    return k


def kernel(x, tok_table, pos_table, gamma, beta):
    B, S = x.shape
    V, D = tok_table.shape
    k = _make_sc_kernel(B, S, V, D)
    out = k(x.reshape(-1), tok_table, pos_table, gamma, beta)
    return out.reshape(B, S, D)


# speedup vs baseline: 1.8464x; 1.8464x over previous
import functools

import jax
import jax.numpy as jnp
from jax import lax
from jax.experimental import pallas as pl
from jax.experimental.pallas import tpu as pltpu
from jax.experimental.pallas import tpu_sc as plsc

L = 16
NC = 2
NS = 16
NW = NC * NS


def _allsum_vec(v):
    idx = lax.iota(jnp.int32, L)
    dnums = lax.GatherDimensionNumbers(
        offset_dims=(), collapsed_slice_dims=(0,), start_index_map=(0,))
    for d in (1, 2, 4, 8):
        v = v + lax.gather(
            v, (idx ^ d)[:, None], dimension_numbers=dnums, slice_sizes=(1,),
            mode=lax.GatherScatterMode.PROMISE_IN_BOUNDS)
    return v


def _rsqrt_vec(v):
    i = lax.bitcast_convert_type(v, jnp.int32)
    magic = jnp.full((L,), 0x5F3759DF, jnp.int32)
    y = lax.bitcast_convert_type(magic - (i >> 1), jnp.float32)
    for _ in range(3):
        y = y * (1.5 - 0.5 * v * y * y)
    return y


def _make_sc_kernel(B, S, V, D):
    bpw = B // NW
    nvec = D // L
    Sp = S + (-S) % 8
    mesh = plsc.VectorSubcoreMesh(
        core_axis_name="c", subcore_axis_name="s", num_cores=NC, num_subcores=NS
    )

    @functools.partial(
        pl.kernel,
        out_type=jax.ShapeDtypeStruct((B, S, D), jnp.float32),
        mesh=mesh,
        compiler_params=pltpu.CompilerParams(use_tc_tiling_on_sc=False),
        scratch_types=[
            pltpu.VMEM((bpw, S), jnp.int32),
            pltpu.VMEM((Sp, D), jnp.float32),
            pltpu.VMEM((D,), jnp.float32),
            pltpu.VMEM((D,), jnp.float32),
            [pltpu.VMEM((S, D), jnp.float32)] * 2,
            pltpu.SMEM((S,), jnp.float32),
            pltpu.SMEM((S,), jnp.float32),
            [pltpu.SemaphoreType.DMA] * 2,
            [pltpu.SemaphoreType.DMA] * 2,
        ],
    )
    def k(x_ref, tok_ref, pos_ref, gamma_ref, beta_ref, out_ref,
          idx_v, pos_v, gamma_v, beta_v, bufs, mean_s, rstd_s, gsems, osems):
        wid = lax.axis_index("s") * NC + lax.axis_index("c")
        base = wid * bpw

        pltpu.sync_copy(x_ref.at[pl.ds(base, bpw)], idx_v)
        pltpu.sync_copy(pos_ref.at[pl.ds(0, Sp)], pos_v)
        pltpu.sync_copy(gamma_ref, gamma_v)
        pltpu.sync_copy(beta_ref, beta_v)

        def issue_gather(c, b):
            pltpu.async_copy(tok_ref.at[idx_v.at[c]], bufs[b], gsems[b])

        issue_gather(0, 0)

        def pass1(buf):
            @plsc.parallel_loop(0, S, unroll=2)
            def row_body(r):
                sacc = [jnp.zeros((L,), jnp.float32) for _ in range(6)]
                qacc = [jnp.zeros((L,), jnp.float32) for _ in range(6)]
                for j in range(nvec):
                    v = buf[r, pl.ds(j * L, L)] + pos_v[r, pl.ds(j * L, L)]
                    buf[r, pl.ds(j * L, L)] = v
                    sacc[j % 6] = sacc[j % 6] + v
                    qacc[j % 6] = qacc[j % 6] + v * v
                s = ((sacc[0] + sacc[1]) + (sacc[2] + sacc[3])) + (sacc[4] + sacc[5])
                q = ((qacc[0] + qacc[1]) + (qacc[2] + qacc[3])) + (qacc[4] + qacc[5])
                mean_v = _allsum_vec(s) * (1.0 / D)
                msq_v = _allsum_vec(q) * (1.0 / D)
                var_v = msq_v - mean_v * mean_v
                rstd_v = _rsqrt_vec(var_v + 1e-5)
                mean_s[r] = mean_v[0]
                rstd_s[r] = rstd_v[0]

        def pass2(buf):
            def col_body(j, _):
                g_vec = gamma_v[pl.ds(j * L, L)]
                b_vec = beta_v[pl.ds(j * L, L)]

                @plsc.parallel_loop(0, S, unroll=2)
                def row_body(r):
                    v = buf[r, pl.ds(j * L, L)]
                    t = (v - mean_s[r]) * rstd_s[r]
                    buf[r, pl.ds(j * L, L)] = t * g_vec + b_vec

                return 0

            lax.fori_loop(0, nvec, col_body, 0)

        def chunk_step(c, b):
            pltpu.make_async_copy(
                tok_ref.at[idx_v.at[c]], bufs[b], gsems[b]).wait()
            pass1(bufs[b])
            @pl.when(c >= 1)
            def _():
                pltpu.make_async_copy(
                    bufs[1 - b], out_ref.at[base], osems[1 - b]).wait()

            @pl.when(c + 1 < bpw)
            def _():
                issue_gather(c + 1, 1 - b)

            pass2(bufs[b])
            pltpu.async_copy(bufs[b], out_ref.at[base + c], osems[b])

        def outer(o, _):
            for b in range(2):
                chunk_step(o * 2 + b, b)
            return 0

        lax.fori_loop(0, bpw // 2, outer, 0)
        b_last = (bpw - 1) % 2
        pltpu.make_async_copy(
            bufs[b_last], out_ref.at[base], osems[b_last]).wait()

    return k


def kernel(x, tok_table, pos_table, gamma, beta):
    B, S = x.shape
    V, D = tok_table.shape
    k = _make_sc_kernel(B, S, V, D)
    return k(x, tok_table, pos_table, gamma, beta)


# speedup vs baseline: 2.6057x; 1.4112x over previous
import functools

import jax
import jax.numpy as jnp
from jax import lax
from jax.experimental import pallas as pl
from jax.experimental.pallas import tpu as pltpu
from jax.experimental.pallas import tpu_sc as plsc

L = 16
NC = 2
NS = 16
NW = NC * NS
BB = 64
PB = 2


def _allsum_vec(v):
    idx = lax.iota(jnp.int32, L)
    dnums = lax.GatherDimensionNumbers(
        offset_dims=(), collapsed_slice_dims=(0,), start_index_map=(0,))
    for d in (1, 2, 4, 8):
        v = v + lax.gather(
            v, (idx ^ d)[:, None], dimension_numbers=dnums, slice_sizes=(1,),
            mode=lax.GatherScatterMode.PROMISE_IN_BOUNDS)
    return v


def _rsqrt_vec(v):
    i = lax.bitcast_convert_type(v, jnp.int32)
    magic = jnp.full((L,), 0x5F3759DF, jnp.int32)
    y = lax.bitcast_convert_type(magic - (i >> 1), jnp.float32)
    for _ in range(3):
        y = y * (1.5 - 0.5 * v * y * y)
    return y


def _make_sc_kernel(B, S, V, D):
    upw = S * PB
    tpw = upw * BB
    nvec = D // L
    mesh = plsc.VectorSubcoreMesh(
        core_axis_name="c", subcore_axis_name="s", num_cores=NC, num_subcores=NS
    )

    @functools.partial(
        pl.kernel,
        out_type=jax.ShapeDtypeStruct((S, B, D), jnp.float32),
        mesh=mesh,
        scratch_types=[
            pltpu.VMEM((tpw,), jnp.int32),
            pltpu.VMEM((D,), jnp.float32),
            pltpu.VMEM((D,), jnp.float32),
            pltpu.VMEM((D,), jnp.float32),
            [pltpu.VMEM((BB, D), jnp.float32)] * 2,
            pltpu.SMEM((BB,), jnp.float32),
            pltpu.SMEM((BB,), jnp.float32),
            [pltpu.SemaphoreType.DMA] * 2,
            [pltpu.SemaphoreType.DMA] * 2,
        ],
    )
    def k(xu_ref, tok_ref, pos_ref, gamma_ref, beta_ref, out_ref,
          idx_v, pos_row, gamma_v, beta_v, bufs, mean_s, rstd_s, gsems, osems):
        wid = lax.axis_index("s") * NC + lax.axis_index("c")

        pltpu.sync_copy(xu_ref.at[pl.ds(wid * tpw, tpw)], idx_v)
        pltpu.sync_copy(gamma_ref, gamma_v)
        pltpu.sync_copy(beta_ref, beta_v)

        def issue_gather(u, b):
            pltpu.async_copy(
                tok_ref.at[idx_v.at[pl.ds(u * BB, BB)]], bufs[b], gsems[b])

        def load_pos(s):
            pltpu.sync_copy(
                pos_ref.at[0, pl.ds(pl.multiple_of(s * D, D), D)], pos_row)

        def pass1(buf):
            @plsc.parallel_loop(0, BB, unroll=2)
            def row_body(r):
                sacc = [jnp.zeros((L,), jnp.float32) for _ in range(6)]
                qacc = [jnp.zeros((L,), jnp.float32) for _ in range(6)]
                for j in range(nvec):
                    v = buf[r, pl.ds(j * L, L)] + pos_row[pl.ds(j * L, L)]
                    buf[r, pl.ds(j * L, L)] = v
                    sacc[j % 6] = sacc[j % 6] + v
                    qacc[j % 6] = qacc[j % 6] + v * v
                s = ((sacc[0] + sacc[1]) + (sacc[2] + sacc[3])) + (sacc[4] + sacc[5])
                q = ((qacc[0] + qacc[1]) + (qacc[2] + qacc[3])) + (qacc[4] + qacc[5])
                mean_v = _allsum_vec(s) * (1.0 / D)
                msq_v = _allsum_vec(q) * (1.0 / D)
                var_v = msq_v - mean_v * mean_v
                rstd_v = _rsqrt_vec(var_v + 1e-5)
                mean_s[r] = mean_v[0]
                rstd_s[r] = rstd_v[0]

        def pass2(buf):
            def col_body(j, _):
                g_vec = gamma_v[pl.ds(j * L, L)]
                b_vec = beta_v[pl.ds(j * L, L)]

                @plsc.parallel_loop(0, BB, unroll=2)
                def row_body(r):
                    v = buf[r, pl.ds(j * L, L)]
                    t = (v - mean_s[r]) * rstd_s[r]
                    buf[r, pl.ds(j * L, L)] = t * g_vec + b_vec

                return 0

            lax.fori_loop(0, nvec, col_body, 0)

        issue_gather(0, 0)

        def s_step(s):
            load_pos(s)
            for b in range(PB):
                u = s * PB + b
                bb = wid * PB + b
                dst = out_ref.at[s, pl.ds(pl.multiple_of(bb * BB, BB), BB)]
                pltpu.make_async_copy(
                    tok_ref.at[idx_v.at[pl.ds(u * BB, BB)]],
                    bufs[b], gsems[b]).wait()
                pass1(bufs[b])
                @pl.when(u >= 1)
                def _():
                    pltpu.make_async_copy(
                        bufs[1 - b], out_ref.at[0, pl.ds(0, BB)],
                        osems[1 - b]).wait()

                @pl.when(u + 1 < upw)
                def _():
                    issue_gather(u + 1, 1 - b)

                pass2(bufs[b])
                pltpu.async_copy(bufs[b], dst, osems[b])

        def outer(s, _):
            s_step(s)
            return 0

        lax.fori_loop(0, S, outer, 0)
        b_last = (upw - 1) % 2
        pltpu.make_async_copy(
            bufs[b_last], out_ref.at[0, pl.ds(0, BB)], osems[b_last]).wait()

    return k


def kernel(x, tok_table, pos_table, gamma, beta):
    B, S = x.shape
    V, D = tok_table.shape
    nbb = B // BB
    xu = (x.T.reshape(S, nbb // PB, PB, BB)
          .transpose(1, 0, 2, 3).reshape(-1))
    pos_flat = pos_table[:S].reshape(1, S * D)
    k = _make_sc_kernel(B, S, V, D)
    out = k(xu, tok_table, pos_flat, gamma, beta)
    return out.transpose(1, 0, 2)
